# baseline (speedup 1.0000x reference)
import functools

import jax
import jax.numpy as jnp
from jax import lax
from jax.experimental import pallas as pl
from jax.experimental.pallas import tpu as pltpu
from jax.experimental.pallas import tpu_sc as plsc

N = 10000
E = 320000
D = 128
DE = 16

NC = 2
NS = 16
NW = NC * NS
CH = 80
WCH = E // (NW * CH)
AGGP = 10240
ROWS_T = AGGP // NS
ROWS_Z = 32
DEGP = 10240
DEG_T = DEGP // NS
DP = D // 2

ROW_BLK = 1000
GRID_N = N // ROW_BLK
EDGE_BLK = 8000
GRID_E = E // EDGE_BLK

_f32 = jnp.float32


def _pack_bf16_pairs(v):
    a = jnp.concatenate([v[:, 32 * k:32 * k + 16] for k in range(4)],
                        axis=1)
    b = jnp.concatenate([v[:, 32 * k + 16:32 * k + 32] for k in range(4)],
                        axis=1)

    def rbits(x):
        u = lax.bitcast_convert_type(x, jnp.uint32)
        return (u + jnp.uint32(0x7FFF) + ((u >> 16) & jnp.uint32(1))) >> 16

    return lax.bitcast_convert_type((rbits(b) << 16) | rbits(a), jnp.int32)


def _node_dense_body(x_ref, g_ref, b_ref, W_ref, Ws_ref, hW_ref, hS_ref):
    x = x_ref[...]
    m = jnp.mean(x, axis=-1, keepdims=True)
    v = jnp.mean((x - m) ** 2, axis=-1, keepdims=True)
    h = (x - m) / jnp.sqrt(v + 1e-5) * g_ref[...] + b_ref[...]
    h = jax.nn.gelu(h)
    hW_ref[...] = jnp.dot(h, W_ref[...], preferred_element_type=_f32)
    hS_ref[...] = jnp.dot(h, Ws_ref[...], preferred_element_type=_f32)


def _gate_body(ea_ref, Wr_ref, gate_ref):
    gate_ref[...] = _pack_bf16_pairs(jax.nn.sigmoid(
        jnp.dot(ea_ref[...], Wr_ref[...], preferred_element_type=_f32)))


def _combine_dense_body(p_ref, deg_ref, hS_ref, b_ref, g_ref, beta_ref,
                        W_ref, Ws_ref, hW_ref, hS2_ref):
    deg = jnp.maximum(deg_ref[0] + deg_ref[1], 1.0)
    out1 = (p_ref[0] + p_ref[1]) / deg + hS_ref[...] + b_ref[...]
    m = jnp.mean(out1, axis=-1, keepdims=True)
    v = jnp.mean((out1 - m) ** 2, axis=-1, keepdims=True)
    h = (out1 - m) / jnp.sqrt(v + 1e-5) * g_ref[...] + beta_ref[...]
    h = jax.nn.gelu(h)
    hW_ref[...] = jnp.dot(h, W_ref[...], preferred_element_type=_f32)
    hS2_ref[...] = jnp.dot(h, Ws_ref[...], preferred_element_type=_f32)


def _final_body(p_ref, deg_ref, hS_ref, b_ref, x_ref, out_ref):
    deg = jnp.maximum(deg_ref[0] + deg_ref[1], 1.0)
    out_ref[...] = ((p_ref[0] + p_ref[1]) / deg
                    + hS_ref[...] + b_ref[...] + x_ref[...])


def _make_edge_body(compute_deg):
    def _edge_body(hw_hbm, gate_hbm, src_hbm, dst_hbm, *refs):
        if compute_deg:
            (agg_out, deg_out, agg_sh, deg_sh, zbuf,
             gate0, gate1, rows0, rows1,
             sidx0, sidx1, didx0, didx1, ones_v, dz,
             gsem0, gsem1, gtsem0, gtsem1, ssem0, ssem1, dsem0, dsem1) = refs
        else:
            (agg_out, agg_sh, zbuf,
             gate0, gate1, rows0, rows1,
             sidx0, sidx1, didx0, didx1,
             gsem0, gsem1, gtsem0, gtsem1, ssem0, ssem1) = refs
        gate = (gate0, gate1)
        rows = (rows0, rows1)
        sidx = (sidx0, sidx1)
        didx = (didx0, didx1)
        gsem = (gsem0, gsem1)
        gtsem = (gtsem0, gtsem1)
        ssem = (ssem0, ssem1)
        if compute_deg:
            dsem = (dsem0, dsem1)

        c = lax.axis_index("c")
        s = lax.axis_index("s")
        wid = c * NS + s
        row0 = s * ROWS_T
        base = wid * WCH

        def _zrow(r, _):
            for k in range(D // 16):
                zbuf[r, pl.ds(k * 16, 16)] = jnp.zeros((16,), _f32)
            return 0
        lax.fori_loop(0, ROWS_Z, _zrow, 0)
        for j in range(ROWS_T // ROWS_Z):
            pltpu.sync_copy(zbuf, agg_sh.at[pl.ds(row0 + j * ROWS_Z,
                                                  ROWS_Z)])
        if compute_deg:
            for k in range(DEG_T // 16):
                dz[pl.ds(k * 16, 16)] = jnp.zeros((16,), _f32)
            for k in range(CH // 16):
                ones_v[pl.ds(k * 16, 16)] = jnp.ones((16,), _f32)
            pltpu.sync_copy(dz, deg_sh.at[pl.ds(s * DEG_T, DEG_T)])
        plsc.subcore_barrier()

        def load_start(ci, b):
            e0 = (base + ci) * CH
            pltpu.sync_copy(src_hbm.at[pl.ds(e0, CH)], sidx[b])
            pltpu.sync_copy(dst_hbm.at[pl.ds(e0, CH)], didx[b])
            pltpu.async_copy(hw_hbm.at[sidx[b]], rows[b], gsem[b])
            pltpu.async_copy(gate_hbm.at[pl.ds(e0, CH)], gate[b], gtsem[b])

        hi_mask = jnp.int32(-65536)

        def compute(b):
            pltpu.make_async_copy(hw_hbm.at[sidx[b]], rows[b],
                                  gsem[b]).wait()
            pltpu.make_async_copy(gate_hbm.at[pl.ds(0, CH)], gate[b],
                                  gtsem[b]).wait()

            def _mrow(r2, _):
                for j in range(2):
                    r = r2 * 2 + j
                    for k in range(D // 32):
                        g32 = gate[b][r, pl.ds(k * 16, 16)]
                        lo_g = plsc.bitcast(lax.shift_left(g32, 16), _f32)
                        hi_g = plsc.bitcast(g32 & hi_mask, _f32)
                        slo = pl.ds(k * 32, 16)
                        shi = pl.ds(k * 32 + 16, 16)
                        rows[b][r, slo] = rows[b][r, slo] * lo_g
                        rows[b][r, shi] = rows[b][r, shi] * hi_g
                return 0
            lax.fori_loop(0, CH // 2, _mrow, 0)
            pltpu.async_copy(rows[b], agg_sh.at[didx[b]], ssem[b],
                             add=True)
            if compute_deg:
                pltpu.async_copy(ones_v, deg_sh.at[didx[b]], dsem[b],
                                 add=True)

        def scatter_wait(b):
            pltpu.make_async_copy(rows[b], agg_sh.at[didx[b]],
                                  ssem[b]).wait()
            if compute_deg:
                pltpu.make_async_copy(ones_v, deg_sh.at[didx[b]],
                                      dsem[b]).wait()

        load_start(0, 0)
        load_start(1, 1)

        def _pair(i2, _):
            compute(0)
            compute(1)
            scatter_wait(0)
            load_start(2 * i2 + 2, 0)

            @pl.when(2 * i2 + 3 < WCH)
            def _():
                scatter_wait(1)
                load_start(2 * i2 + 3, 1)
            return 0
        lax.fori_loop(0, (WCH - 1) // 2, _pair, 0)
        compute(0)
        scatter_wait(1)
        scatter_wait(0)
        plsc.subcore_barrier()

        for j in range(ROWS_T // ROWS_Z):
            r0 = row0 + j * ROWS_Z
            pltpu.sync_copy(agg_sh.at[pl.ds(r0, ROWS_Z)], zbuf)
            pltpu.sync_copy(zbuf, agg_out.at[c, pl.ds(r0, ROWS_Z)])
        if compute_deg:
            pltpu.sync_copy(deg_sh.at[pl.ds(s * DEG_T, DEG_T)], dz)
            pltpu.sync_copy(dz, deg_out.at[c, pl.ds(s * DEG_T, DEG_T)])
    return _edge_body


def _make_edge_pass(compute_deg):
    out_type = [jax.ShapeDtypeStruct((NC, AGGP, D), _f32)]
    scratch = [pltpu.VMEM_SHARED((AGGP, D), _f32)]
    if compute_deg:
        out_type.append(jax.ShapeDtypeStruct((NC, DEGP), _f32))
        scratch.append(pltpu.VMEM_SHARED((DEGP,), _f32))
    scratch += [
        pltpu.VMEM((ROWS_Z, D), _f32),
        pltpu.VMEM((CH, DP), jnp.int32),
        pltpu.VMEM((CH, DP), jnp.int32),
        pltpu.VMEM((CH, D), _f32),
        pltpu.VMEM((CH, D), _f32),
        pltpu.VMEM((CH,), jnp.int32),
        pltpu.VMEM((CH,), jnp.int32),
        pltpu.VMEM((CH,), jnp.int32),
        pltpu.VMEM((CH,), jnp.int32),
    ]
    if compute_deg:
        scratch += [
            pltpu.VMEM((CH,), _f32),
            pltpu.VMEM((DEG_T,), _f32),
        ]
    nsem = 8 if compute_deg else 6
    scratch += [pltpu.SemaphoreType.DMA] * nsem
    return pl.kernel(
        _make_edge_body(compute_deg),
        out_type=tuple(out_type),
        mesh=plsc.VectorSubcoreMesh(core_axis_name="c",
                                    subcore_axis_name="s"),
        scratch_types=scratch,
        compiler_params=pltpu.CompilerParams(needs_layout_passes=False),
    )


_edge_pass_deg = _make_edge_pass(True)
_edge_pass_nodeg = _make_edge_pass(False)


def _node_dense(x, g, beta, W, Ws):
    return pl.pallas_call(
        _node_dense_body,
        grid=(GRID_N,),
        in_specs=[
            pl.BlockSpec((ROW_BLK, D), lambda i: (i, 0)),
            pl.BlockSpec((1, D), lambda i: (0, 0)),
            pl.BlockSpec((1, D), lambda i: (0, 0)),
            pl.BlockSpec((D, D), lambda i: (0, 0)),
            pl.BlockSpec((D, D), lambda i: (0, 0)),
        ],
        out_specs=[pl.BlockSpec((ROW_BLK, D), lambda i: (i, 0)),
                   pl.BlockSpec((ROW_BLK, D), lambda i: (i, 0))],
        out_shape=[jax.ShapeDtypeStruct((N, D), _f32),
                   jax.ShapeDtypeStruct((N, D), _f32)],
    )(x, g.reshape(1, D), beta.reshape(1, D), W, Ws)


def _gate(edge_attr, Wr):
    return pl.pallas_call(
        _gate_body,
        grid=(GRID_E,),
        in_specs=[
            pl.BlockSpec((EDGE_BLK, DE), lambda i: (i, 0)),
            pl.BlockSpec((DE, D), lambda i: (0, 0)),
        ],
        out_specs=pl.BlockSpec((EDGE_BLK, DP), lambda i: (i, 0)),
        out_shape=jax.ShapeDtypeStruct((E, DP), jnp.int32),
    )(edge_attr, Wr)


def _combine_dense(p, degp, hS, b, g, beta, W, Ws):
    return pl.pallas_call(
        _combine_dense_body,
        grid=(GRID_N,),
        in_specs=[
            pl.BlockSpec((NC, ROW_BLK, D), lambda i: (0, i, 0)),
            pl.BlockSpec((NC, ROW_BLK, 1), lambda i: (0, i, 0)),
            pl.BlockSpec((ROW_BLK, D), lambda i: (i, 0)),
            pl.BlockSpec((1, D), lambda i: (0, 0)),
            pl.BlockSpec((1, D), lambda i: (0, 0)),
            pl.BlockSpec((1, D), lambda i: (0, 0)),
            pl.BlockSpec((D, D), lambda i: (0, 0)),
            pl.BlockSpec((D, D), lambda i: (0, 0)),
        ],
        out_specs=[pl.BlockSpec((ROW_BLK, D), lambda i: (i, 0)),
                   pl.BlockSpec((ROW_BLK, D), lambda i: (i, 0))],
        out_shape=[jax.ShapeDtypeStruct((N, D), _f32),
                   jax.ShapeDtypeStruct((N, D), _f32)],
    )(p, degp, hS, b.reshape(1, D), g.reshape(1, D), beta.reshape(1, D),
      W, Ws)


def _final(p, degp, hS, b, x):
    return pl.pallas_call(
        _final_body,
        grid=(GRID_N,),
        in_specs=[
            pl.BlockSpec((NC, ROW_BLK, D), lambda i: (0, i, 0)),
            pl.BlockSpec((NC, ROW_BLK, 1), lambda i: (0, i, 0)),
            pl.BlockSpec((ROW_BLK, D), lambda i: (i, 0)),
            pl.BlockSpec((1, D), lambda i: (0, 0)),
            pl.BlockSpec((ROW_BLK, D), lambda i: (i, 0)),
        ],
        out_specs=pl.BlockSpec((ROW_BLK, D), lambda i: (i, 0)),
        out_shape=jax.ShapeDtypeStruct((N, D), _f32),
    )(p, degp, hS, b.reshape(1, D), x)


def kernel(x, edge_index, edge_attr, g1, beta1, W1, Wr1, Ws1, b1,
           g2, beta2, W2, Wr2, Ws2, b2):
    src = edge_index[0]
    dst = edge_index[1]

    hW1, hS1 = _node_dense(x, g1, beta1, W1, Ws1)
    gate1 = _gate(edge_attr, Wr1)
    p1, deg1 = _edge_pass_deg(hW1, gate1, src, dst)
    deg1 = deg1.reshape(NC, DEGP, 1)

    hW2, hS2 = _combine_dense(p1, deg1, hS1, b1, g2, beta2, W2, Ws2)
    gate2 = _gate(edge_attr, Wr2)
    (p2,) = _edge_pass_nodeg(hW2, gate2, src, dst)

    return _final(p2, deg1, hS2, b2, x)


# speedup vs baseline: 1.0791x; 1.0791x over previous
import functools

import jax
import jax.numpy as jnp
from jax import lax
from jax.experimental import pallas as pl
from jax.experimental.pallas import tpu as pltpu
from jax.experimental.pallas import tpu_sc as plsc

N = 10000
E = 320000
D = 128
DE = 16

NC = 2
NS = 16
NW = NC * NS
CH = 80
WCH = E // (NW * CH)
AGGP = 10240
ROWS_T = AGGP // NS
ROWS_Z = 32
DEGP = 10240
DEG_T = DEGP // NS

ROW_BLK = 1000
GRID_N = N // ROW_BLK
EDGE_BLK = 16000
GRID_E = E // EDGE_BLK

_f32 = jnp.float32


def _node_dense_body(x_ref, g_ref, b_ref, W_ref, Ws_ref, hW_ref, hS_ref):
    x = x_ref[...]
    m = jnp.mean(x, axis=-1, keepdims=True)
    v = jnp.mean((x - m) ** 2, axis=-1, keepdims=True)
    h = (x - m) / jnp.sqrt(v + 1e-5) * g_ref[...] + b_ref[...]
    h = jax.nn.gelu(h)
    hW_ref[...] = jnp.dot(h, W_ref[...], preferred_element_type=_f32)
    hS_ref[...] = jnp.dot(h, Ws_ref[...], preferred_element_type=_f32)


def _gate_body(ea_ref, Wr_ref, gate_ref):
    gate_ref[...] = jax.nn.sigmoid(
        jnp.dot(ea_ref[...], Wr_ref[...], preferred_element_type=_f32))


def _combine_dense_body(p_ref, deg_ref, hS_ref, b_ref, g_ref, beta_ref,
                        W_ref, Ws_ref, hW_ref, hS2_ref):
    deg = jnp.maximum(deg_ref[0] + deg_ref[1], 1.0)
    out1 = (p_ref[0] + p_ref[1]) / deg + hS_ref[...] + b_ref[...]
    m = jnp.mean(out1, axis=-1, keepdims=True)
    v = jnp.mean((out1 - m) ** 2, axis=-1, keepdims=True)
    h = (out1 - m) / jnp.sqrt(v + 1e-5) * g_ref[...] + beta_ref[...]
    h = jax.nn.gelu(h)
    hW_ref[...] = jnp.dot(h, W_ref[...], preferred_element_type=_f32)
    hS2_ref[...] = jnp.dot(h, Ws_ref[...], preferred_element_type=_f32)


def _final_body(p_ref, deg_ref, hS_ref, b_ref, x_ref, out_ref):
    deg = jnp.maximum(deg_ref[0] + deg_ref[1], 1.0)
    out_ref[...] = ((p_ref[0] + p_ref[1]) / deg
                    + hS_ref[...] + b_ref[...] + x_ref[...])


def _make_edge_body(compute_deg):
    def _edge_body(hw_hbm, gate_hbm, src_hbm, dst_hbm, *refs):
        if compute_deg:
            (agg_out, deg_out, agg_sh, deg_sh, zbuf,
             gate0, gate1, rows0, rows1, sidx0, sidx1, didx0, didx1,
             ones_v, dz,
             gsem0, gsem1, gtsem0, gtsem1, ssem0, ssem1, dsem0, dsem1) = refs
        else:
            (agg_out, agg_sh, zbuf,
             gate0, gate1, rows0, rows1, sidx0, sidx1, didx0, didx1,
             gsem0, gsem1, gtsem0, gtsem1, ssem0, ssem1) = refs
        gate = (gate0, gate1)
        rows = (rows0, rows1)
        sidx = (sidx0, sidx1)
        didx = (didx0, didx1)
        gsem = (gsem0, gsem1)
        gtsem = (gtsem0, gtsem1)
        ssem = (ssem0, ssem1)
        if compute_deg:
            dsem = (dsem0, dsem1)

        c = lax.axis_index("c")
        s = lax.axis_index("s")
        wid = c * NS + s
        row0 = s * ROWS_T
        base = wid * WCH

        def _zrow(r, _):
            for k in range(D // 16):
                zbuf[r, pl.ds(k * 16, 16)] = jnp.zeros((16,), _f32)
            return 0
        lax.fori_loop(0, ROWS_Z, _zrow, 0)
        for j in range(ROWS_T // ROWS_Z):
            pltpu.sync_copy(zbuf, agg_sh.at[pl.ds(row0 + j * ROWS_Z,
                                                  ROWS_Z)])
        if compute_deg:
            for k in range(DEG_T // 16):
                dz[pl.ds(k * 16, 16)] = jnp.zeros((16,), _f32)
            for k in range(CH // 16):
                ones_v[pl.ds(k * 16, 16)] = jnp.ones((16,), _f32)
            pltpu.sync_copy(dz, deg_sh.at[pl.ds(s * DEG_T, DEG_T)])
        plsc.subcore_barrier()

        def load_start(ci, b):
            e0 = (base + ci) * CH
            pltpu.sync_copy(src_hbm.at[pl.ds(e0, CH)], sidx[b])
            pltpu.sync_copy(dst_hbm.at[pl.ds(e0, CH)], didx[b])
            pltpu.async_copy(hw_hbm.at[sidx[b]], rows[b], gsem[b])
            pltpu.async_copy(gate_hbm.at[pl.ds(e0, CH)], gate[b], gtsem[b])

        def compute(b):
            pltpu.make_async_copy(hw_hbm.at[sidx[b]], rows[b],
                                  gsem[b]).wait()
            pltpu.make_async_copy(gate_hbm.at[pl.ds(0, CH)], gate[b],
                                  gtsem[b]).wait()

            def _mrow(r4, _):
                for j in range(4):
                    r = r4 * 4 + j
                    for k in range(D // 16):
                        sl = pl.ds(k * 16, 16)
                        rows[b][r, sl] = rows[b][r, sl] * gate[b][r, sl]
                return 0
            lax.fori_loop(0, CH // 4, _mrow, 0)
            pltpu.async_copy(rows[b], agg_sh.at[didx[b]], ssem[b],
                             add=True)
            if compute_deg:
                pltpu.async_copy(ones_v, deg_sh.at[didx[b]], dsem[b],
                                 add=True)

        def scatter_wait(b):
            pltpu.make_async_copy(rows[b], agg_sh.at[didx[b]],
                                  ssem[b]).wait()
            if compute_deg:
                pltpu.make_async_copy(ones_v, deg_sh.at[didx[b]],
                                      dsem[b]).wait()

        load_start(0, 0)
        load_start(1, 1)

        def _pair(i2, _):
            compute(0)
            compute(1)
            scatter_wait(0)
            load_start(2 * i2 + 2, 0)

            @pl.when(2 * i2 + 3 < WCH)
            def _():
                scatter_wait(1)
                load_start(2 * i2 + 3, 1)
            return 0
        lax.fori_loop(0, (WCH - 1) // 2, _pair, 0)
        compute(0)
        scatter_wait(1)
        scatter_wait(0)
        plsc.subcore_barrier()

        for j in range(ROWS_T // ROWS_Z):
            r0 = row0 + j * ROWS_Z
            pltpu.sync_copy(agg_sh.at[pl.ds(r0, ROWS_Z)], zbuf)
            pltpu.sync_copy(zbuf, agg_out.at[c, pl.ds(r0, ROWS_Z)])
        if compute_deg:
            pltpu.sync_copy(deg_sh.at[pl.ds(s * DEG_T, DEG_T)], dz)
            pltpu.sync_copy(dz, deg_out.at[c, pl.ds(s * DEG_T, DEG_T)])
    return _edge_body


def _make_edge_pass(compute_deg):
    out_type = [jax.ShapeDtypeStruct((NC, AGGP, D), _f32)]
    scratch = [pltpu.VMEM_SHARED((AGGP, D), _f32)]
    if compute_deg:
        out_type.append(jax.ShapeDtypeStruct((NC, DEGP), _f32))
        scratch.append(pltpu.VMEM_SHARED((DEGP,), _f32))
    scratch += [
        pltpu.VMEM((ROWS_Z, D), _f32),
        pltpu.VMEM((CH, D), _f32),
        pltpu.VMEM((CH, D), _f32),
        pltpu.VMEM((CH, D), _f32),
        pltpu.VMEM((CH, D), _f32),
        pltpu.VMEM((CH,), jnp.int32),
        pltpu.VMEM((CH,), jnp.int32),
        pltpu.VMEM((CH,), jnp.int32),
        pltpu.VMEM((CH,), jnp.int32),
    ]
    if compute_deg:
        scratch += [
            pltpu.VMEM((CH,), _f32),
            pltpu.VMEM((DEG_T,), _f32),
        ]
    nsem = 8 if compute_deg else 6
    scratch += [pltpu.SemaphoreType.DMA] * nsem
    return pl.kernel(
        _make_edge_body(compute_deg),
        out_type=tuple(out_type),
        mesh=plsc.VectorSubcoreMesh(core_axis_name="c",
                                    subcore_axis_name="s"),
        scratch_types=scratch,
    )


_edge_pass_deg = _make_edge_pass(True)
_edge_pass_nodeg = _make_edge_pass(False)


def _node_dense(x, g, beta, W, Ws):
    return pl.pallas_call(
        _node_dense_body,
        grid=(GRID_N,),
        in_specs=[
            pl.BlockSpec((ROW_BLK, D), lambda i: (i, 0)),
            pl.BlockSpec((1, D), lambda i: (0, 0)),
            pl.BlockSpec((1, D), lambda i: (0, 0)),
            pl.BlockSpec((D, D), lambda i: (0, 0)),
            pl.BlockSpec((D, D), lambda i: (0, 0)),
        ],
        out_specs=[pl.BlockSpec((ROW_BLK, D), lambda i: (i, 0))] * 2,
        out_shape=[jax.ShapeDtypeStruct((N, D), _f32)] * 2,
    )(x, g.reshape(1, D), beta.reshape(1, D), W, Ws)


def _gate(edge_attr, Wr):
    return pl.pallas_call(
        _gate_body,
        grid=(GRID_E,),
        in_specs=[
            pl.BlockSpec((EDGE_BLK, DE), lambda i: (i, 0)),
            pl.BlockSpec((DE, D), lambda i: (0, 0)),
        ],
        out_specs=pl.BlockSpec((EDGE_BLK, D), lambda i: (i, 0)),
        out_shape=jax.ShapeDtypeStruct((E, D), _f32),
    )(edge_attr, Wr)


def _combine_dense(p, degp, hS, b, g, beta, W, Ws):
    return pl.pallas_call(
        _combine_dense_body,
        grid=(GRID_N,),
        in_specs=[
            pl.BlockSpec((NC, ROW_BLK, D), lambda i: (0, i, 0)),
            pl.BlockSpec((NC, ROW_BLK, 1), lambda i: (0, i, 0)),
            pl.BlockSpec((ROW_BLK, D), lambda i: (i, 0)),
            pl.BlockSpec((1, D), lambda i: (0, 0)),
            pl.BlockSpec((1, D), lambda i: (0, 0)),
            pl.BlockSpec((1, D), lambda i: (0, 0)),
            pl.BlockSpec((D, D), lambda i: (0, 0)),
            pl.BlockSpec((D, D), lambda i: (0, 0)),
        ],
        out_specs=[pl.BlockSpec((ROW_BLK, D), lambda i: (i, 0))] * 2,
        out_shape=[jax.ShapeDtypeStruct((N, D), _f32)] * 2,
    )(p, degp, hS, b.reshape(1, D), g.reshape(1, D), beta.reshape(1, D),
      W, Ws)


def _final(p, degp, hS, b, x):
    return pl.pallas_call(
        _final_body,
        grid=(GRID_N,),
        in_specs=[
            pl.BlockSpec((NC, ROW_BLK, D), lambda i: (0, i, 0)),
            pl.BlockSpec((NC, ROW_BLK, 1), lambda i: (0, i, 0)),
            pl.BlockSpec((ROW_BLK, D), lambda i: (i, 0)),
            pl.BlockSpec((1, D), lambda i: (0, 0)),
            pl.BlockSpec((ROW_BLK, D), lambda i: (i, 0)),
        ],
        out_specs=pl.BlockSpec((ROW_BLK, D), lambda i: (i, 0)),
        out_shape=jax.ShapeDtypeStruct((N, D), _f32),
    )(p, degp, hS, b.reshape(1, D), x)


def kernel(x, edge_index, edge_attr, g1, beta1, W1, Wr1, Ws1, b1,
           g2, beta2, W2, Wr2, Ws2, b2):
    src = edge_index[0]
    dst = edge_index[1]

    hW1, hS1 = _node_dense(x, g1, beta1, W1, Ws1)
    gate1 = _gate(edge_attr, Wr1)
    p1, deg1 = _edge_pass_deg(hW1, gate1, src, dst)
    deg1 = deg1.reshape(NC, DEGP, 1)

    hW2, hS2 = _combine_dense(p1, deg1, hS1, b1, g2, beta2, W2, Ws2)
    gate2 = _gate(edge_attr, Wr2)
    (p2,) = _edge_pass_nodeg(hW2, gate2, src, dst)

    return _final(p2, deg1, hS2, b2, x)
